# Initial kernel scaffold; baseline (speedup 1.0000x reference)
import jax
import jax.numpy as jnp
from jax.experimental import pallas as pl


def kernel(x, edge_index, W0l, W0r, b0, W1l, W1r, b1, Wlin, blin):
    raise NotImplementedError("write your pallas kernel here")


# speedup vs baseline: 7.1564x; 7.1564x over previous
import functools

import jax
import jax.numpy as jnp
from jax import lax
from jax.experimental import pallas as pl
from jax.experimental.pallas import tpu as pltpu
from jax.experimental.pallas import tpu_sc as plsc

N = 10000
E = 320000
D = 128

NC = 2
NS = 16
L = 16
NW = NC * NS
EW = E // NW
CH = 80
NCHUNK = EW // CH
IDXB = 5
ROFF = 624
RLEN = 640
ZR = 40

_mesh = plsc.VectorSubcoreMesh(core_axis_name="c", subcore_axis_name="s")


def _make_agg(want_deg):
    out_types = [jax.ShapeDtypeStruct((NC, N, D), jnp.float32)]
    if want_deg:
        out_types.append(jax.ShapeDtypeStruct((NC * N,), jnp.float32))
    scratch = [
        pltpu.VMEM((IDXB, CH), jnp.int32),
        pltpu.VMEM((IDXB, CH), jnp.int32),
        pltpu.VMEM((CH, D), jnp.float32),
        pltpu.VMEM((ZR, D), jnp.float32),
        pltpu.VMEM_SHARED((N, D), jnp.float32),
        pltpu.SemaphoreType.DMA,
    ]
    if want_deg:
        scratch += [
            pltpu.VMEM((RLEN,), jnp.float32),
            pltpu.VMEM((CH,), jnp.float32),
            pltpu.VMEM_SHARED((N,), jnp.float32),
        ]

    @functools.partial(
        pl.kernel, mesh=_mesh,
        out_type=tuple(out_types) if want_deg else out_types[0],
        scratch_types=scratch)
    def agg_kernel(edge_hbm, x_hbm, *refs):
        if want_deg:
            (agg_out, deg_out, src_v, dst_v, rows_v, z2d, acc_sh, sem,
             zdeg, ones_v, deg_sh) = refs
        else:
            agg_out, src_v, dst_v, rows_v, z2d, acc_sh, sem = refs
        cid = lax.axis_index("c")
        sid = lax.axis_index("s")
        wid = sid * NC + cid

        def zfill(i, _):
            z2d[i // (D // L), pl.ds((i % (D // L)) * L, L)] = jnp.zeros(
                (L,), jnp.float32)
            return 0
        lax.fori_loop(0, ZR * (D // L), zfill, 0)
        if want_deg:
            def zdfill(i, _):
                zdeg[pl.ds(i * L, L)] = jnp.zeros((L,), jnp.float32)
                return 0
            lax.fori_loop(0, RLEN // L, zdfill, 0)
            for q in range(CH // L):
                ones_v[pl.ds(q * L, L)] = jnp.ones((L,), jnp.float32)

        roff = pl.multiple_of(sid * ROFF, 8)
        for q in range(RLEN // ZR):
            pltpu.sync_copy(z2d, acc_sh.at[pl.ds(pl.multiple_of(
                sid * ROFF + q * ZR, 8), ZR)])
        if want_deg:
            pltpu.sync_copy(zdeg, deg_sh.at[pl.ds(roff, RLEN)])

        plsc.subcore_barrier()

        def superchunk(g, _):
            pltpu.sync_copy(edge_hbm.at[0, wid, g], src_v)
            pltpu.sync_copy(edge_hbm.at[1, wid, g], dst_v)
            for j in range(IDXB):
                pltpu.async_copy(x_hbm.at[src_v.at[j]], rows_v, sem).wait()
                pltpu.sync_copy(rows_v, acc_sh.at[dst_v.at[j]], add=True)
                if want_deg:
                    pltpu.sync_copy(ones_v, deg_sh.at[dst_v.at[j]], add=True)
            return 0
        lax.fori_loop(0, NCHUNK // IDXB, superchunk, 0)

        plsc.subcore_barrier()

        pltpu.sync_copy(acc_sh.at[pl.ds(roff, RLEN)],
                        agg_out.at[cid, pl.ds(roff, RLEN)])
        if want_deg:
            pltpu.sync_copy(deg_sh.at[pl.ds(roff, RLEN)], zdeg)
            pltpu.sync_copy(zdeg, deg_out.at[pl.ds(pl.multiple_of(
                cid * N + sid * ROFF, 8), RLEN)])

    return agg_kernel


_agg_deg = _make_agg(True)
_agg = _make_agg(False)

R = 1000


def _mm_t(a, w):
    return lax.dot_general(a, w, (((1,), (1,)), ((), ())),
                           preferred_element_type=jnp.float32)


def _conv0_body(x_ref, a_ref, d_ref, wl_ref, wr_ref, b_ref, o_ref):
    inv = 1.0 / jnp.maximum(d_ref[0] + d_ref[1], 1.0)
    mean = (a_ref[0] + a_ref[1]) * inv
    h = _mm_t(x_ref[...], wl_ref[...]) + _mm_t(mean, wr_ref[...]) + b_ref[...]
    o_ref[...] = jnp.maximum(h, 0.0)


def _conv1_lin_body(x_ref, a_ref, d_ref, wl_ref, wr_ref, b_ref, wlin_ref,
                    blin_ref, o_ref):
    inv = 1.0 / jnp.maximum(d_ref[0] + d_ref[1], 1.0)
    mean = (a_ref[0] + a_ref[1]) * inv
    h = _mm_t(x_ref[...], wl_ref[...]) + _mm_t(mean, wr_ref[...]) + b_ref[...]
    o_ref[...] = _mm_t(h, wlin_ref[...]) + blin_ref[...]


_row_spec = pl.BlockSpec((R, D), lambda i: (i, 0))
_agg_spec = pl.BlockSpec((2, R, D), lambda i: (0, i, 0))
_deg_spec = pl.BlockSpec((2, R, 1), lambda i: (0, i, 0))
_w_spec = pl.BlockSpec((D, D), lambda i: (0, 0))
_b_spec = pl.BlockSpec((1, D), lambda i: (0, 0))


def _conv0_tc(x, aggp, degp3, Wl, Wr, b):
    return pl.pallas_call(
        _conv0_body,
        grid=(N // R,),
        in_specs=[_row_spec, _agg_spec, _deg_spec, _w_spec, _w_spec, _b_spec],
        out_specs=_row_spec,
        out_shape=jax.ShapeDtypeStruct((N, D), jnp.float32),
    )(x, aggp, degp3, Wl, Wr, b)


def _conv1_lin_tc(h1, aggp, degp3, Wl, Wr, b, Wlin, blin):
    return pl.pallas_call(
        _conv1_lin_body,
        grid=(N // R,),
        in_specs=[_row_spec, _agg_spec, _deg_spec, _w_spec, _w_spec, _b_spec,
                  _w_spec, _b_spec],
        out_specs=_row_spec,
        out_shape=jax.ShapeDtypeStruct((N, D), jnp.float32),
    )(h1, aggp, degp3, Wl, Wr, b, Wlin, blin)


def kernel(x, edge_index, W0l, W0r, b0, W1l, W1r, b1, Wlin, blin):
    edge_r = edge_index.reshape(2, NW, NCHUNK // IDXB, IDXB, CH)
    aggp0, degp = _agg_deg(edge_r, x)
    degp3 = degp.reshape(NC, N, 1)
    h1 = _conv0_tc(x, aggp0, degp3, W0l, W0r, b0.reshape(1, D))
    aggp1 = _agg(edge_r, h1)
    return _conv1_lin_tc(h1, aggp1, degp3, W1l, W1r, b1.reshape(1, D),
                         Wlin, blin.reshape(1, D))


# speedup vs baseline: 10.8567x; 1.5171x over previous
import functools

import jax
import jax.numpy as jnp
from jax import lax
from jax.experimental import pallas as pl
from jax.experimental.pallas import tpu as pltpu
from jax.experimental.pallas import tpu_sc as plsc

N = 10000
E = 320000
D = 128

NC = 2
NS = 16
L = 16
NW = NC * NS
EW = E // NW
CH = 100
NCHUNK = EW // CH
NHALF = NCHUNK // 2
ONESB = 112
ROFF = 624
RLEN = 640
ZR = 8

_mesh = plsc.VectorSubcoreMesh(core_axis_name="c", subcore_axis_name="s")


def _make_agg(want_deg):
    out_types = [jax.ShapeDtypeStruct((NC, N, D), jnp.float32)]
    if want_deg:
        out_types.append(jax.ShapeDtypeStruct((NC * N,), jnp.float32))
    scratch = [
        pltpu.VMEM((NHALF, CH), jnp.int32),
        pltpu.VMEM((NHALF, CH), jnp.int32),
        pltpu.VMEM((CH, D), jnp.float32),
        pltpu.VMEM((CH, D), jnp.float32),
        pltpu.VMEM((ZR, D), jnp.float32),
        pltpu.VMEM_SHARED((N, D), jnp.float32),
        pltpu.SemaphoreType.DMA,
        pltpu.SemaphoreType.DMA,
    ]
    if want_deg:
        scratch += [
            pltpu.VMEM((RLEN,), jnp.float32),
            pltpu.VMEM((ONESB,), jnp.float32),
            pltpu.VMEM_SHARED((N,), jnp.float32),
        ]

    @functools.partial(
        pl.kernel, mesh=_mesh,
        out_type=tuple(out_types) if want_deg else out_types[0],
        scratch_types=scratch)
    def agg_kernel(edge_hbm, x_hbm, *refs):
        if want_deg:
            (agg_out, deg_out, src_v, dst_v, rows_a, rows_b, z2d, acc_sh,
             gsem_a, gsem_b, zdeg, ones_v, deg_sh) = refs
        else:
            (agg_out, src_v, dst_v, rows_a, rows_b, z2d, acc_sh,
             gsem_a, gsem_b) = refs
        cid = lax.axis_index("c")
        sid = lax.axis_index("s")
        wid = sid * NC + cid
        rows = (rows_a, rows_b)
        gsem = (gsem_a, gsem_b)

        def gather(j, p):
            return pltpu.make_async_copy(x_hbm.at[src_v.at[j]], rows[p],
                                         gsem[p])

        def zfill(i, _):
            z2d[i // (D // L), pl.ds((i % (D // L)) * L, L)] = jnp.zeros(
                (L,), jnp.float32)
            return 0
        lax.fori_loop(0, ZR * (D // L), zfill, 0)
        if want_deg:
            def zdfill(i, _):
                zdeg[pl.ds(i * L, L)] = jnp.zeros((L,), jnp.float32)
                return 0
            lax.fori_loop(0, RLEN // L, zdfill, 0)
            for q in range(ONESB // L):
                ones_v[pl.ds(q * L, L)] = jnp.ones((L,), jnp.float32)

        roff = pl.multiple_of(sid * ROFF, 8)
        def zacc(q, _):
            pltpu.sync_copy(z2d, acc_sh.at[pl.ds(pl.multiple_of(
                sid * ROFF + q * ZR, 8), ZR)])
            return 0
        lax.fori_loop(0, RLEN // ZR, zacc, 0)
        if want_deg:
            pltpu.sync_copy(zdeg, deg_sh.at[pl.ds(roff, RLEN)])

        plsc.subcore_barrier()

        def scat(j, p):
            pltpu.sync_copy(rows[p], acc_sh.at[dst_v.at[j]], add=True)
            if want_deg:
                pltpu.sync_copy(ones_v.at[pl.ds(0, CH)],
                                deg_sh.at[dst_v.at[j]], add=True)

        def chunk2(t, _):
            j = t * 2
            gather(j, 0).wait()
            gather(j + 1, 1).start()
            scat(j, 0)
            gather(j + 1, 1).wait()
            gather(j + 2, 0).start()
            scat(j + 1, 1)
            return 0

        for h in range(2):
            pltpu.sync_copy(edge_hbm.at[0, wid, h], src_v)
            pltpu.sync_copy(edge_hbm.at[1, wid, h], dst_v)
            gather(0, 0).start()
            lax.fori_loop(0, NHALF // 2 - 1, chunk2, 0)
            j = NHALF - 2
            gather(j, 0).wait()
            gather(j + 1, 1).start()
            scat(j, 0)
            gather(j + 1, 1).wait()
            scat(j + 1, 1)

        plsc.subcore_barrier()

        pltpu.sync_copy(acc_sh.at[pl.ds(roff, RLEN)],
                        agg_out.at[cid, pl.ds(roff, RLEN)])
        if want_deg:
            pltpu.sync_copy(deg_sh.at[pl.ds(roff, RLEN)], zdeg)
            pltpu.sync_copy(zdeg, deg_out.at[pl.ds(pl.multiple_of(
                cid * N + sid * ROFF, 8), RLEN)])

    return agg_kernel


_agg_deg = _make_agg(True)
_agg = _make_agg(False)

R = 1000


def _mm_t(a, w):
    return lax.dot_general(a, w, (((1,), (1,)), ((), ())),
                           preferred_element_type=jnp.float32)


def _conv0_body(x_ref, a_ref, d_ref, wl_ref, wr_ref, b_ref, o_ref):
    inv = 1.0 / jnp.maximum(d_ref[0] + d_ref[1], 1.0)
    mean = (a_ref[0] + a_ref[1]) * inv
    h = _mm_t(x_ref[...], wl_ref[...]) + _mm_t(mean, wr_ref[...]) + b_ref[...]
    o_ref[...] = jnp.maximum(h, 0.0)


def _conv1_lin_body(x_ref, a_ref, d_ref, wl_ref, wr_ref, b_ref, wlin_ref,
                    blin_ref, o_ref):
    inv = 1.0 / jnp.maximum(d_ref[0] + d_ref[1], 1.0)
    mean = (a_ref[0] + a_ref[1]) * inv
    h = _mm_t(x_ref[...], wl_ref[...]) + _mm_t(mean, wr_ref[...]) + b_ref[...]
    o_ref[...] = _mm_t(h, wlin_ref[...]) + blin_ref[...]


_row_spec = pl.BlockSpec((R, D), lambda i: (i, 0))
_agg_spec = pl.BlockSpec((2, R, D), lambda i: (0, i, 0))
_deg_spec = pl.BlockSpec((2, R, 1), lambda i: (0, i, 0))
_w_spec = pl.BlockSpec((D, D), lambda i: (0, 0))
_b_spec = pl.BlockSpec((1, D), lambda i: (0, 0))


def _conv0_tc(x, aggp, degp3, Wl, Wr, b):
    return pl.pallas_call(
        _conv0_body,
        grid=(N // R,),
        in_specs=[_row_spec, _agg_spec, _deg_spec, _w_spec, _w_spec, _b_spec],
        out_specs=_row_spec,
        out_shape=jax.ShapeDtypeStruct((N, D), jnp.float32),
    )(x, aggp, degp3, Wl, Wr, b)


def _conv1_lin_tc(h1, aggp, degp3, Wl, Wr, b, Wlin, blin):
    return pl.pallas_call(
        _conv1_lin_body,
        grid=(N // R,),
        in_specs=[_row_spec, _agg_spec, _deg_spec, _w_spec, _w_spec, _b_spec,
                  _w_spec, _b_spec],
        out_specs=_row_spec,
        out_shape=jax.ShapeDtypeStruct((N, D), jnp.float32),
    )(h1, aggp, degp3, Wl, Wr, b, Wlin, blin)


def kernel(x, edge_index, W0l, W0r, b0, W1l, W1r, b1, Wlin, blin):
    edge_r = edge_index.reshape(2, NW, 2, NHALF, CH)
    aggp0, degp = _agg_deg(edge_r, x)
    degp3 = degp.reshape(NC, N, 1)
    h1 = _conv0_tc(x, aggp0, degp3, W0l, W0r, b0.reshape(1, D))
    aggp1 = _agg(edge_r, h1)
    return _conv1_lin_tc(h1, aggp1, degp3, W1l, W1r, b1.reshape(1, D),
                         Wlin, blin.reshape(1, D))


# speedup vs baseline: 10.9062x; 1.0046x over previous
import functools

import jax
import jax.numpy as jnp
from jax import lax
from jax.experimental import pallas as pl
from jax.experimental.pallas import tpu as pltpu
from jax.experimental.pallas import tpu_sc as plsc

N = 10000
E = 320000
D = 128

NC = 2
NS = 16
L = 16
NW = NC * NS
EW = E // NW
CH = 100
NCHUNK = EW // CH
NHALF = NCHUNK // 2
ONESB = 112
ROFF = 624
RLEN = 640
ZR = 8

_mesh = plsc.VectorSubcoreMesh(core_axis_name="c", subcore_axis_name="s")


def _make_agg(want_deg):
    out_types = [jax.ShapeDtypeStruct((NC, N, D), jnp.float32)]
    if want_deg:
        out_types.append(jax.ShapeDtypeStruct((NC * N,), jnp.float32))
    scratch = [
        pltpu.VMEM((NHALF, CH), jnp.int32),
        pltpu.VMEM((NHALF, CH), jnp.int32),
        pltpu.VMEM((CH, D), jnp.float32),
        pltpu.VMEM((CH, D), jnp.float32),
        pltpu.VMEM((ZR, D), jnp.float32),
        pltpu.VMEM_SHARED((N, D), jnp.float32),
        pltpu.SemaphoreType.DMA,
        pltpu.SemaphoreType.DMA,
        pltpu.SemaphoreType.DMA,
        pltpu.SemaphoreType.DMA,
    ]
    if want_deg:
        scratch += [
            pltpu.VMEM((RLEN,), jnp.float32),
            pltpu.VMEM((ONESB,), jnp.float32),
            pltpu.VMEM_SHARED((N,), jnp.float32),
            pltpu.SemaphoreType.DMA,
        ]

    @functools.partial(
        pl.kernel, mesh=_mesh,
        out_type=tuple(out_types) if want_deg else out_types[0],
        scratch_types=scratch)
    def agg_kernel(edge_hbm, x_hbm, *refs):
        if want_deg:
            (agg_out, deg_out, src_v, dst_v, rows_a, rows_b, z2d, acc_sh,
             gsem_a, gsem_b, ssem_a, ssem_b, zdeg, ones_v, deg_sh,
             dsem) = refs
        else:
            (agg_out, src_v, dst_v, rows_a, rows_b, z2d, acc_sh,
             gsem_a, gsem_b, ssem_a, ssem_b) = refs
        cid = lax.axis_index("c")
        sid = lax.axis_index("s")
        wid = sid * NC + cid
        rows = (rows_a, rows_b)
        gsem = (gsem_a, gsem_b)
        ssem = (ssem_a, ssem_b)

        def gather(j, p):
            return pltpu.make_async_copy(x_hbm.at[src_v.at[j]], rows[p],
                                         gsem[p])

        def scatter(j, p):
            return pltpu.make_async_copy(rows[p], acc_sh.at[dst_v.at[j]],
                                         ssem[p])

        def zfill(i, _):
            z2d[i // (D // L), pl.ds((i % (D // L)) * L, L)] = jnp.zeros(
                (L,), jnp.float32)
            return 0
        lax.fori_loop(0, ZR * (D // L), zfill, 0)
        if want_deg:
            def zdfill(i, _):
                zdeg[pl.ds(i * L, L)] = jnp.zeros((L,), jnp.float32)
                return 0
            lax.fori_loop(0, RLEN // L, zdfill, 0)
            for q in range(ONESB // L):
                ones_v[pl.ds(q * L, L)] = jnp.ones((L,), jnp.float32)

        roff = pl.multiple_of(sid * ROFF, 8)
        def zacc(q, _):
            pltpu.sync_copy(z2d, acc_sh.at[pl.ds(pl.multiple_of(
                sid * ROFF + q * ZR, 8), ZR)])
            return 0
        lax.fori_loop(0, RLEN // ZR, zacc, 0)
        if want_deg:
            pltpu.sync_copy(zdeg, deg_sh.at[pl.ds(roff, RLEN)])

        plsc.subcore_barrier()

        def scat_deg(j):
            if want_deg:
                pltpu.make_async_copy(
                    ones_v.at[pl.ds(0, CH)], deg_sh.at[dst_v.at[j]],
                    dsem).start(add=True)

        def chunk2(t, _):
            j = t * 2
            gather(j, 0).wait()
            scatter(j, 0).start(add=True)
            scat_deg(j)
            gather(j + 1, 1).wait()
            scatter(j + 1, 1).start(add=True)
            scat_deg(j + 1)
            scatter(j, 0).wait()
            gather(j + 2, 0).start()
            scatter(j + 1, 1).wait()
            gather(j + 3, 1).start()
            return 0

        for h in range(2):
            pltpu.sync_copy(edge_hbm.at[0, wid, h], src_v)
            pltpu.sync_copy(edge_hbm.at[1, wid, h], dst_v)
            gather(0, 0).start()
            gather(1, 1).start()
            lax.fori_loop(0, NHALF // 2 - 1, chunk2, 0)
            j = NHALF - 2
            gather(j, 0).wait()
            scatter(j, 0).start(add=True)
            scat_deg(j)
            gather(j + 1, 1).wait()
            scatter(j + 1, 1).start(add=True)
            scat_deg(j + 1)
            scatter(j, 0).wait()
            scatter(j + 1, 1).wait()
            if want_deg:
                def ddrain(i, _):
                    pltpu.make_async_copy(
                        ones_v.at[pl.ds(0, CH)], deg_sh.at[dst_v.at[i]],
                        dsem).wait()
                    return 0
                lax.fori_loop(0, NHALF, ddrain, 0)

        plsc.subcore_barrier()

        pltpu.sync_copy(acc_sh.at[pl.ds(roff, RLEN)],
                        agg_out.at[cid, pl.ds(roff, RLEN)])
        if want_deg:
            pltpu.sync_copy(deg_sh.at[pl.ds(roff, RLEN)], zdeg)
            pltpu.sync_copy(zdeg, deg_out.at[pl.ds(pl.multiple_of(
                cid * N + sid * ROFF, 8), RLEN)])

    return agg_kernel


_agg_deg = _make_agg(True)
_agg = _make_agg(False)

R = 1000


def _mm_t(a, w):
    return lax.dot_general(a, w, (((1,), (1,)), ((), ())),
                           preferred_element_type=jnp.float32)


def _conv0_body(x_ref, a_ref, d_ref, wl_ref, wr_ref, b_ref, o_ref):
    inv = 1.0 / jnp.maximum(d_ref[0] + d_ref[1], 1.0)
    mean = (a_ref[0] + a_ref[1]) * inv
    h = _mm_t(x_ref[...], wl_ref[...]) + _mm_t(mean, wr_ref[...]) + b_ref[...]
    o_ref[...] = jnp.maximum(h, 0.0)


def _conv1_lin_body(x_ref, a_ref, d_ref, wl_ref, wr_ref, b_ref, wlin_ref,
                    blin_ref, o_ref):
    inv = 1.0 / jnp.maximum(d_ref[0] + d_ref[1], 1.0)
    mean = (a_ref[0] + a_ref[1]) * inv
    h = _mm_t(x_ref[...], wl_ref[...]) + _mm_t(mean, wr_ref[...]) + b_ref[...]
    o_ref[...] = _mm_t(h, wlin_ref[...]) + blin_ref[...]


_row_spec = pl.BlockSpec((R, D), lambda i: (i, 0))
_agg_spec = pl.BlockSpec((2, R, D), lambda i: (0, i, 0))
_deg_spec = pl.BlockSpec((2, R, 1), lambda i: (0, i, 0))
_w_spec = pl.BlockSpec((D, D), lambda i: (0, 0))
_b_spec = pl.BlockSpec((1, D), lambda i: (0, 0))


def _conv0_tc(x, aggp, degp3, Wl, Wr, b):
    return pl.pallas_call(
        _conv0_body,
        grid=(N // R,),
        in_specs=[_row_spec, _agg_spec, _deg_spec, _w_spec, _w_spec, _b_spec],
        out_specs=_row_spec,
        out_shape=jax.ShapeDtypeStruct((N, D), jnp.float32),
    )(x, aggp, degp3, Wl, Wr, b)


def _conv1_lin_tc(h1, aggp, degp3, Wl, Wr, b, Wlin, blin):
    return pl.pallas_call(
        _conv1_lin_body,
        grid=(N // R,),
        in_specs=[_row_spec, _agg_spec, _deg_spec, _w_spec, _w_spec, _b_spec,
                  _w_spec, _b_spec],
        out_specs=_row_spec,
        out_shape=jax.ShapeDtypeStruct((N, D), jnp.float32),
    )(h1, aggp, degp3, Wl, Wr, b, Wlin, blin)


def kernel(x, edge_index, W0l, W0r, b0, W1l, W1r, b1, Wlin, blin):
    edge_r = edge_index.reshape(2, NW, 2, NHALF, CH)
    aggp0, degp = _agg_deg(edge_r, x)
    degp3 = degp.reshape(NC, N, 1)
    h1 = _conv0_tc(x, aggp0, degp3, W0l, W0r, b0.reshape(1, D))
    aggp1 = _agg(edge_r, h1)
    return _conv1_lin_tc(h1, aggp1, degp3, W1l, W1r, b1.reshape(1, D),
                         Wlin, blin.reshape(1, D))


# speedup vs baseline: 11.0706x; 1.0151x over previous
import functools

import jax
import jax.numpy as jnp
from jax import lax
from jax.experimental import pallas as pl
from jax.experimental.pallas import tpu as pltpu
from jax.experimental.pallas import tpu_sc as plsc

N = 10000
E = 320000
D = 128

NC = 2
NS = 16
L = 16
NW = NC * NS
EW = E // NW
CH = 125
NCHUNK = EW // CH
NHALF = NCHUNK // 2
ONESB = 128
ROFF = 624
RLEN = 640
ZR = 8

_mesh = plsc.VectorSubcoreMesh(core_axis_name="c", subcore_axis_name="s")


def _make_agg(want_deg):
    out_types = [jax.ShapeDtypeStruct((NC, N, D), jnp.float32)]
    if want_deg:
        out_types.append(jax.ShapeDtypeStruct((NC * N,), jnp.float32))
    scratch = [
        pltpu.VMEM((NHALF, CH), jnp.int32),
        pltpu.VMEM((NHALF, CH), jnp.int32),
        pltpu.VMEM((CH, D), jnp.float32),
        pltpu.VMEM((CH, D), jnp.float32),
        pltpu.VMEM((ZR, D), jnp.float32),
        pltpu.VMEM_SHARED((N, D), jnp.float32),
        pltpu.SemaphoreType.DMA,
        pltpu.SemaphoreType.DMA,
        pltpu.SemaphoreType.DMA,
        pltpu.SemaphoreType.DMA,
    ]
    if want_deg:
        scratch += [
            pltpu.VMEM((RLEN,), jnp.float32),
            pltpu.VMEM((ONESB,), jnp.float32),
            pltpu.VMEM_SHARED((N,), jnp.float32),
            pltpu.SemaphoreType.DMA,
        ]

    @functools.partial(
        pl.kernel, mesh=_mesh,
        out_type=tuple(out_types) if want_deg else out_types[0],
        scratch_types=scratch)
    def agg_kernel(edge_hbm, x_hbm, *refs):
        if want_deg:
            (agg_out, deg_out, src_v, dst_v, rows_a, rows_b, z2d, acc_sh,
             gsem_a, gsem_b, ssem_a, ssem_b, zdeg, ones_v, deg_sh,
             dsem) = refs
        else:
            (agg_out, src_v, dst_v, rows_a, rows_b, z2d, acc_sh,
             gsem_a, gsem_b, ssem_a, ssem_b) = refs
        cid = lax.axis_index("c")
        sid = lax.axis_index("s")
        wid = sid * NC + cid
        rows = (rows_a, rows_b)
        gsem = (gsem_a, gsem_b)
        ssem = (ssem_a, ssem_b)

        def gather(j, p):
            return pltpu.make_async_copy(x_hbm.at[src_v.at[j]], rows[p],
                                         gsem[p])

        def scatter(j, p):
            return pltpu.make_async_copy(rows[p], acc_sh.at[dst_v.at[j]],
                                         ssem[p])

        def zfill(i, _):
            z2d[i // (D // L), pl.ds((i % (D // L)) * L, L)] = jnp.zeros(
                (L,), jnp.float32)
            return 0
        lax.fori_loop(0, ZR * (D // L), zfill, 0)
        if want_deg:
            def zdfill(i, _):
                zdeg[pl.ds(i * L, L)] = jnp.zeros((L,), jnp.float32)
                return 0
            lax.fori_loop(0, RLEN // L, zdfill, 0)
            for q in range(ONESB // L):
                ones_v[pl.ds(q * L, L)] = jnp.ones((L,), jnp.float32)

        roff = pl.multiple_of(sid * ROFF, 8)
        def zacc(q, _):
            pltpu.sync_copy(z2d, acc_sh.at[pl.ds(pl.multiple_of(
                sid * ROFF + q * ZR, 8), ZR)])
            return 0
        lax.fori_loop(0, RLEN // ZR, zacc, 0)
        if want_deg:
            pltpu.sync_copy(zdeg, deg_sh.at[pl.ds(roff, RLEN)])

        plsc.subcore_barrier()

        def scat_deg(j):
            if want_deg:
                pltpu.make_async_copy(
                    ones_v.at[pl.ds(0, CH)], deg_sh.at[dst_v.at[j]],
                    dsem).start(add=True)

        def chunk2(t, _):
            j = t * 2
            gather(j, 0).wait()
            scatter(j, 0).start(add=True)
            scat_deg(j)
            gather(j + 1, 1).wait()
            scatter(j + 1, 1).start(add=True)
            scat_deg(j + 1)
            scatter(j, 0).wait()
            gather(j + 2, 0).start()
            scatter(j + 1, 1).wait()
            gather(j + 3, 1).start()
            return 0

        for h in range(2):
            pltpu.sync_copy(edge_hbm.at[0, wid, h], src_v)
            pltpu.sync_copy(edge_hbm.at[1, wid, h], dst_v)
            gather(0, 0).start()
            gather(1, 1).start()
            lax.fori_loop(0, NHALF // 2 - 1, chunk2, 0)
            j = NHALF - 2
            gather(j, 0).wait()
            scatter(j, 0).start(add=True)
            scat_deg(j)
            gather(j + 1, 1).wait()
            scatter(j + 1, 1).start(add=True)
            scat_deg(j + 1)
            scatter(j, 0).wait()
            scatter(j + 1, 1).wait()
            if want_deg:
                def ddrain(i, _):
                    pltpu.make_async_copy(
                        ones_v.at[pl.ds(0, CH)], deg_sh.at[dst_v.at[i]],
                        dsem).wait()
                    return 0
                lax.fori_loop(0, NHALF, ddrain, 0)

        plsc.subcore_barrier()

        pltpu.sync_copy(acc_sh.at[pl.ds(roff, RLEN)],
                        agg_out.at[cid, pl.ds(roff, RLEN)])
        if want_deg:
            pltpu.sync_copy(deg_sh.at[pl.ds(roff, RLEN)], zdeg)
            pltpu.sync_copy(zdeg, deg_out.at[pl.ds(pl.multiple_of(
                cid * N + sid * ROFF, 8), RLEN)])

    return agg_kernel


_agg_deg = _make_agg(True)
_agg = _make_agg(False)

R = 1000


def _mm_t(a, w):
    return lax.dot_general(a, w, (((1,), (1,)), ((), ())),
                           preferred_element_type=jnp.float32)


def _conv0_body(x_ref, a_ref, d_ref, wl_ref, wr_ref, b_ref, o_ref):
    inv = 1.0 / jnp.maximum(d_ref[0] + d_ref[1], 1.0)
    mean = (a_ref[0] + a_ref[1]) * inv
    h = _mm_t(x_ref[...], wl_ref[...]) + _mm_t(mean, wr_ref[...]) + b_ref[...]
    o_ref[...] = jnp.maximum(h, 0.0)


def _conv1_lin_body(x_ref, a_ref, d_ref, wl_ref, wr_ref, b_ref, wlin_ref,
                    blin_ref, o_ref):
    inv = 1.0 / jnp.maximum(d_ref[0] + d_ref[1], 1.0)
    mean = (a_ref[0] + a_ref[1]) * inv
    h = _mm_t(x_ref[...], wl_ref[...]) + _mm_t(mean, wr_ref[...]) + b_ref[...]
    o_ref[...] = _mm_t(h, wlin_ref[...]) + blin_ref[...]


_row_spec = pl.BlockSpec((R, D), lambda i: (i, 0))
_agg_spec = pl.BlockSpec((2, R, D), lambda i: (0, i, 0))
_deg_spec = pl.BlockSpec((2, R, 1), lambda i: (0, i, 0))
_w_spec = pl.BlockSpec((D, D), lambda i: (0, 0))
_b_spec = pl.BlockSpec((1, D), lambda i: (0, 0))


def _conv0_tc(x, aggp, degp3, Wl, Wr, b):
    return pl.pallas_call(
        _conv0_body,
        grid=(N // R,),
        in_specs=[_row_spec, _agg_spec, _deg_spec, _w_spec, _w_spec, _b_spec],
        out_specs=_row_spec,
        out_shape=jax.ShapeDtypeStruct((N, D), jnp.float32),
    )(x, aggp, degp3, Wl, Wr, b)


def _conv1_lin_tc(h1, aggp, degp3, Wl, Wr, b, Wlin, blin):
    return pl.pallas_call(
        _conv1_lin_body,
        grid=(N // R,),
        in_specs=[_row_spec, _agg_spec, _deg_spec, _w_spec, _w_spec, _b_spec,
                  _w_spec, _b_spec],
        out_specs=_row_spec,
        out_shape=jax.ShapeDtypeStruct((N, D), jnp.float32),
    )(h1, aggp, degp3, Wl, Wr, b, Wlin, blin)


def kernel(x, edge_index, W0l, W0r, b0, W1l, W1r, b1, Wlin, blin):
    edge_r = edge_index.reshape(2, NW, 2, NHALF, CH)
    aggp0, degp = _agg_deg(edge_r, x)
    degp3 = degp.reshape(NC, N, 1)
    h1 = _conv0_tc(x, aggp0, degp3, W0l, W0r, b0.reshape(1, D))
    aggp1 = _agg(edge_r, h1)
    return _conv1_lin_tc(h1, aggp1, degp3, W1l, W1r, b1.reshape(1, D),
                         Wlin, blin.reshape(1, D))


# speedup vs baseline: 11.2394x; 1.0152x over previous
import functools

import jax
import jax.numpy as jnp
from jax import lax
from jax.experimental import pallas as pl
from jax.experimental.pallas import tpu as pltpu
from jax.experimental.pallas import tpu_sc as plsc

N = 10000
E = 320000
D = 128

NC = 2
NS = 16
L = 16
NW = NC * NS
EW = E // NW
CH = 125
NCHUNK = EW // CH
NHALF = NCHUNK // 2
ONESB = 128
ROFF = 624
RLEN = 640
ZR = 8

_mesh = plsc.VectorSubcoreMesh(core_axis_name="c", subcore_axis_name="s")


def _make_agg(want_deg):
    out_types = [jax.ShapeDtypeStruct((NC, N, D), jnp.float32)]
    if want_deg:
        out_types.append(jax.ShapeDtypeStruct((NC * N,), jnp.float32))
    scratch = [
        pltpu.VMEM((NHALF, CH), jnp.int32),
        pltpu.VMEM((NHALF, CH), jnp.int32),
        pltpu.VMEM((CH, D), jnp.float32),
        pltpu.VMEM((CH, D), jnp.float32),
        pltpu.VMEM((ZR, D), jnp.float32),
        pltpu.VMEM_SHARED((N, D), jnp.float32),
        pltpu.SemaphoreType.DMA,
        pltpu.SemaphoreType.DMA,
        pltpu.SemaphoreType.DMA,
        pltpu.SemaphoreType.DMA,
    ]
    if want_deg:
        scratch += [
            pltpu.VMEM((RLEN,), jnp.float32),
            pltpu.VMEM((ONESB,), jnp.float32),
            pltpu.VMEM_SHARED((N,), jnp.float32),
            pltpu.SemaphoreType.DMA,
        ]

    @functools.partial(
        pl.kernel, mesh=_mesh,
        out_type=tuple(out_types) if want_deg else out_types[0],
        scratch_types=scratch)
    def agg_kernel(edge_hbm, x_hbm, *refs):
        if want_deg:
            (agg_out, deg_out, src_v, dst_v, rows_a, rows_b, z2d, acc_sh,
             gsem_a, gsem_b, ssem_a, ssem_b, zdeg, ones_v, deg_sh,
             dsem) = refs
        else:
            (agg_out, src_v, dst_v, rows_a, rows_b, z2d, acc_sh,
             gsem_a, gsem_b, ssem_a, ssem_b) = refs
        cid = lax.axis_index("c")
        sid = lax.axis_index("s")
        wid = sid * NC + cid
        rows = (rows_a, rows_b)
        gsem = (gsem_a, gsem_b)
        ssem = (ssem_a, ssem_b)

        def gather(j, p):
            return pltpu.make_async_copy(x_hbm.at[src_v.at[j]], rows[p],
                                         gsem[p])

        def scatter(j, p):
            return pltpu.make_async_copy(rows[p], acc_sh.at[dst_v.at[j]],
                                         ssem[p])

        def zfill(i, _):
            z2d[i // (D // L), pl.ds((i % (D // L)) * L, L)] = jnp.zeros(
                (L,), jnp.float32)
            return 0
        lax.fori_loop(0, ZR * (D // L), zfill, 0)
        if want_deg:
            def zdfill(i, _):
                zdeg[pl.ds(i * L, L)] = jnp.zeros((L,), jnp.float32)
                return 0
            lax.fori_loop(0, RLEN // L, zdfill, 0)
            for q in range(ONESB // L):
                ones_v[pl.ds(q * L, L)] = jnp.ones((L,), jnp.float32)

        roff = pl.multiple_of(sid * ROFF, 8)
        def zacc(q, _):
            pltpu.sync_copy(z2d, acc_sh.at[pl.ds(pl.multiple_of(
                sid * ROFF + q * ZR, 8), ZR)])
            return 0
        lax.fori_loop(0, RLEN // ZR, zacc, 0)
        if want_deg:
            pltpu.sync_copy(zdeg, deg_sh.at[pl.ds(roff, RLEN)])

        plsc.subcore_barrier()

        def scat_deg(j):
            if want_deg:
                pltpu.make_async_copy(
                    ones_v.at[pl.ds(0, CH)], deg_sh.at[dst_v.at[j]],
                    dsem).start(add=True)

        def chunk2(t, _):
            j = t * 2
            gather(j, 0).wait()
            scatter(j, 0).start(add=True)
            scat_deg(j)
            gather(j + 1, 1).wait()
            scatter(j + 1, 1).start(add=True)
            scat_deg(j + 1)
            scatter(j, 0).wait()
            gather(j + 2, 0).start()
            scatter(j + 1, 1).wait()
            gather(j + 3, 1).start()
            return 0

        for h in range(2):
            pltpu.sync_copy(edge_hbm.at[0, wid, h], src_v)
            pltpu.sync_copy(edge_hbm.at[1, wid, h], dst_v)
            gather(0, 0).start()
            gather(1, 1).start()
            lax.fori_loop(0, NHALF // 2 - 1, chunk2, 0)
            j = NHALF - 2
            gather(j, 0).wait()
            scatter(j, 0).start(add=True)
            scat_deg(j)
            gather(j + 1, 1).wait()
            scatter(j + 1, 1).start(add=True)
            scat_deg(j + 1)
            scatter(j, 0).wait()
            scatter(j + 1, 1).wait()
            if want_deg:
                def ddrain(i, _):
                    pltpu.make_async_copy(
                        ones_v.at[pl.ds(0, CH)], deg_sh.at[dst_v.at[i]],
                        dsem).wait()
                    return 0
                lax.fori_loop(0, NHALF, ddrain, 0)

        plsc.subcore_barrier()

        pltpu.sync_copy(acc_sh.at[pl.ds(roff, RLEN)],
                        agg_out.at[cid, pl.ds(roff, RLEN)])
        if want_deg:
            pltpu.sync_copy(deg_sh.at[pl.ds(roff, RLEN)], zdeg)
            pltpu.sync_copy(zdeg, deg_out.at[pl.ds(pl.multiple_of(
                cid * N + sid * ROFF, 8), RLEN)])

    return agg_kernel


_agg_deg = _make_agg(True)
_agg = _make_agg(False)

R = 2000


def _mm_t(a, w):
    return lax.dot_general(a, w, (((1,), (1,)), ((), ())),
                           preferred_element_type=jnp.float32)


def _conv0_body(x_ref, a_ref, d_ref, wl_ref, wr_ref, b_ref, o_ref):
    inv = 1.0 / jnp.maximum(d_ref[0] + d_ref[1], 1.0)
    mean = (a_ref[0] + a_ref[1]) * inv
    h = _mm_t(x_ref[...], wl_ref[...]) + _mm_t(mean, wr_ref[...]) + b_ref[...]
    o_ref[...] = jnp.maximum(h, 0.0)


def _conv1_lin_body(x_ref, a_ref, d_ref, wl_ref, wr_ref, b_ref, wlin_ref,
                    blin_ref, o_ref):
    inv = 1.0 / jnp.maximum(d_ref[0] + d_ref[1], 1.0)
    mean = (a_ref[0] + a_ref[1]) * inv
    h = _mm_t(x_ref[...], wl_ref[...]) + _mm_t(mean, wr_ref[...]) + b_ref[...]
    o_ref[...] = _mm_t(h, wlin_ref[...]) + blin_ref[...]


_row_spec = pl.BlockSpec((R, D), lambda i: (i, 0))
_agg_spec = pl.BlockSpec((2, R, D), lambda i: (0, i, 0))
_deg_spec = pl.BlockSpec((2, R, 1), lambda i: (0, i, 0))
_w_spec = pl.BlockSpec((D, D), lambda i: (0, 0))
_b_spec = pl.BlockSpec((1, D), lambda i: (0, 0))


def _conv0_tc(x, aggp, degp3, Wl, Wr, b):
    return pl.pallas_call(
        _conv0_body,
        grid=(N // R,),
        in_specs=[_row_spec, _agg_spec, _deg_spec, _w_spec, _w_spec, _b_spec],
        out_specs=_row_spec,
        out_shape=jax.ShapeDtypeStruct((N, D), jnp.float32),
    )(x, aggp, degp3, Wl, Wr, b)


def _conv1_lin_tc(h1, aggp, degp3, Wl, Wr, b, Wlin, blin):
    return pl.pallas_call(
        _conv1_lin_body,
        grid=(N // R,),
        in_specs=[_row_spec, _agg_spec, _deg_spec, _w_spec, _w_spec, _b_spec,
                  _w_spec, _b_spec],
        out_specs=_row_spec,
        out_shape=jax.ShapeDtypeStruct((N, D), jnp.float32),
    )(h1, aggp, degp3, Wl, Wr, b, Wlin, blin)


def kernel(x, edge_index, W0l, W0r, b0, W1l, W1r, b1, Wlin, blin):
    edge_r = edge_index.reshape(2, NW, 2, NHALF, CH)
    aggp0, degp = _agg_deg(edge_r, x)
    degp3 = degp.reshape(NC, N, 1)
    h1 = _conv0_tc(x, aggp0, degp3, W0l, W0r, b0.reshape(1, D))
    aggp1 = _agg(edge_r, h1)
    return _conv1_lin_tc(h1, aggp1, degp3, W1l, W1r, b1.reshape(1, D),
                         Wlin, blin.reshape(1, D))


# speedup vs baseline: 11.2498x; 1.0009x over previous
import functools

import jax
import jax.numpy as jnp
from jax import lax
from jax.experimental import pallas as pl
from jax.experimental.pallas import tpu as pltpu
from jax.experimental.pallas import tpu_sc as plsc

N = 10000
E = 320000
D = 128

NC = 2
NS = 16
L = 16
NW = NC * NS
EW = E // NW
CH = 125
NCHUNK = EW // CH
NHALF = NCHUNK // 2
ONESB = 128
ROFF = 624
RLEN = 640
ZR = 8

_mesh = plsc.VectorSubcoreMesh(core_axis_name="c", subcore_axis_name="s")


def _make_agg(want_deg):
    out_types = [jax.ShapeDtypeStruct((NC, N, D), jnp.float32)]
    if want_deg:
        out_types.append(jax.ShapeDtypeStruct((NC * N,), jnp.float32))
    scratch = [
        pltpu.VMEM((NHALF, CH), jnp.int32),
        pltpu.VMEM((NHALF, CH), jnp.int32),
        pltpu.VMEM((CH, D), jnp.float32),
        pltpu.VMEM((CH, D), jnp.float32),
        pltpu.VMEM((ZR, D), jnp.float32),
        pltpu.VMEM_SHARED((N, D), jnp.float32),
        pltpu.SemaphoreType.DMA,
        pltpu.SemaphoreType.DMA,
        pltpu.SemaphoreType.DMA,
        pltpu.SemaphoreType.DMA,
    ]
    if want_deg:
        scratch += [
            pltpu.VMEM((RLEN,), jnp.float32),
            pltpu.VMEM((ONESB,), jnp.float32),
            pltpu.VMEM_SHARED((N,), jnp.float32),
            pltpu.SemaphoreType.DMA,
        ]

    @functools.partial(
        pl.kernel, mesh=_mesh,
        out_type=tuple(out_types) if want_deg else out_types[0],
        scratch_types=scratch)
    def agg_kernel(edge_hbm, x_hbm, *refs):
        if want_deg:
            (agg_out, deg_out, src_v, dst_v, rows_a, rows_b, z2d, acc_sh,
             gsem_a, gsem_b, ssem_a, ssem_b, zdeg, ones_v, deg_sh,
             dsem) = refs
        else:
            (agg_out, src_v, dst_v, rows_a, rows_b, z2d, acc_sh,
             gsem_a, gsem_b, ssem_a, ssem_b) = refs
        cid = lax.axis_index("c")
        sid = lax.axis_index("s")
        wid = sid * NC + cid
        rows = (rows_a, rows_b)
        gsem = (gsem_a, gsem_b)
        ssem = (ssem_a, ssem_b)

        def gather(j, p):
            return pltpu.make_async_copy(x_hbm.at[src_v.at[j]], rows[p],
                                         gsem[p])

        def scatter(j, p):
            return pltpu.make_async_copy(rows[p], acc_sh.at[dst_v.at[j]],
                                         ssem[p])

        def zfill(i, _):
            z2d[i // (D // L), pl.ds((i % (D // L)) * L, L)] = jnp.zeros(
                (L,), jnp.float32)
            return 0
        lax.fori_loop(0, ZR * (D // L), zfill, 0)
        if want_deg:
            def zdfill(i, _):
                zdeg[pl.ds(i * L, L)] = jnp.zeros((L,), jnp.float32)
                return 0
            lax.fori_loop(0, RLEN // L, zdfill, 0)
            for q in range(ONESB // L):
                ones_v[pl.ds(q * L, L)] = jnp.ones((L,), jnp.float32)

        roff = pl.multiple_of(sid * ROFF, 8)
        def zacc(q, _):
            pltpu.sync_copy(z2d, acc_sh.at[pl.ds(pl.multiple_of(
                sid * ROFF + q * ZR, 8), ZR)])
            return 0
        lax.fori_loop(0, RLEN // ZR, zacc, 0)
        if want_deg:
            pltpu.sync_copy(zdeg, deg_sh.at[pl.ds(roff, RLEN)])

        plsc.subcore_barrier()

        def scat_deg(j):
            if want_deg:
                pltpu.make_async_copy(
                    ones_v.at[pl.ds(0, CH)], deg_sh.at[dst_v.at[j]],
                    dsem).start(add=True)

        def chunk2(t, _):
            j = t * 2
            gather(j, 0).wait()
            scatter(j, 0).start(add=True)
            scat_deg(j)
            gather(j + 1, 1).wait()
            scatter(j + 1, 1).start(add=True)
            scat_deg(j + 1)
            scatter(j, 0).wait()
            gather(j + 2, 0).start()
            scatter(j + 1, 1).wait()
            gather(j + 3, 1).start()
            return 0

        for h in range(2):
            pltpu.sync_copy(edge_hbm.at[0, wid, h], src_v)
            pltpu.sync_copy(edge_hbm.at[1, wid, h], dst_v)
            gather(0, 0).start()
            gather(1, 1).start()
            lax.fori_loop(0, NHALF // 2 - 1, chunk2, 0)
            j = NHALF - 2
            gather(j, 0).wait()
            scatter(j, 0).start(add=True)
            scat_deg(j)
            gather(j + 1, 1).wait()
            scatter(j + 1, 1).start(add=True)
            scat_deg(j + 1)
            scatter(j, 0).wait()
            scatter(j + 1, 1).wait()
            if want_deg:
                def ddrain(i, _):
                    pltpu.make_async_copy(
                        ones_v.at[pl.ds(0, CH)], deg_sh.at[dst_v.at[i]],
                        dsem).wait()
                    return 0
                lax.fori_loop(0, NHALF, ddrain, 0)

        plsc.subcore_barrier()

        pltpu.sync_copy(acc_sh.at[pl.ds(roff, RLEN)],
                        agg_out.at[cid, pl.ds(roff, RLEN)])
        if want_deg:
            pltpu.sync_copy(deg_sh.at[pl.ds(roff, RLEN)], zdeg)
            pltpu.sync_copy(zdeg, deg_out.at[pl.ds(pl.multiple_of(
                cid * N + sid * ROFF, 8), RLEN)])

    return agg_kernel


_agg_deg = _make_agg(True)
_agg = _make_agg(False)

R = 2000


def _mm_t(a, w):
    return lax.dot_general(a, w, (((1,), (1,)), ((), ())),
                           preferred_element_type=jnp.float32)


def _mm_body(x_ref, w_ref, o_ref):
    o_ref[...] = _mm_t(x_ref[...], w_ref[...])


def _mm_tc(x, w):
    return pl.pallas_call(
        _mm_body,
        grid=(N // R,),
        in_specs=[_row_spec, _w_spec],
        out_specs=_row_spec,
        out_shape=jax.ShapeDtypeStruct((N, D), jnp.float32),
    )(x, w)


def _conv0_body(xl_ref, a_ref, d_ref, wr_ref, b_ref, o_ref):
    inv = 1.0 / jnp.maximum(d_ref[0] + d_ref[1], 1.0)
    mean = (a_ref[0] + a_ref[1]) * inv
    h = xl_ref[...] + _mm_t(mean, wr_ref[...]) + b_ref[...]
    o_ref[...] = jnp.maximum(h, 0.0)


def _conv1_lin_body(x_ref, a_ref, d_ref, wl_ref, wr_ref, b_ref, wlin_ref,
                    blin_ref, o_ref):
    inv = 1.0 / jnp.maximum(d_ref[0] + d_ref[1], 1.0)
    mean = (a_ref[0] + a_ref[1]) * inv
    h = _mm_t(x_ref[...], wl_ref[...]) + _mm_t(mean, wr_ref[...]) + b_ref[...]
    o_ref[...] = _mm_t(h, wlin_ref[...]) + blin_ref[...]


_row_spec = pl.BlockSpec((R, D), lambda i: (i, 0))
_agg_spec = pl.BlockSpec((2, R, D), lambda i: (0, i, 0))
_deg_spec = pl.BlockSpec((2, R, 1), lambda i: (0, i, 0))
_w_spec = pl.BlockSpec((D, D), lambda i: (0, 0))
_b_spec = pl.BlockSpec((1, D), lambda i: (0, 0))


def _conv0_tc(xl, aggp, degp3, Wr, b):
    return pl.pallas_call(
        _conv0_body,
        grid=(N // R,),
        in_specs=[_row_spec, _agg_spec, _deg_spec, _w_spec, _b_spec],
        out_specs=_row_spec,
        out_shape=jax.ShapeDtypeStruct((N, D), jnp.float32),
    )(xl, aggp, degp3, Wr, b)


def _conv1_lin_tc(h1, aggp, degp3, Wl, Wr, b, Wlin, blin):
    return pl.pallas_call(
        _conv1_lin_body,
        grid=(N // R,),
        in_specs=[_row_spec, _agg_spec, _deg_spec, _w_spec, _w_spec, _b_spec,
                  _w_spec, _b_spec],
        out_specs=_row_spec,
        out_shape=jax.ShapeDtypeStruct((N, D), jnp.float32),
    )(h1, aggp, degp3, Wl, Wr, b, Wlin, blin)


def kernel(x, edge_index, W0l, W0r, b0, W1l, W1r, b1, Wlin, blin):
    edge_r = edge_index.reshape(2, NW, 2, NHALF, CH)
    aggp0, degp = _agg_deg(edge_r, x)
    xl = _mm_tc(x, W0l)
    degp3 = degp.reshape(NC, N, 1)
    h1 = _conv0_tc(xl, aggp0, degp3, W0r, b0.reshape(1, D))
    aggp1 = _agg(edge_r, h1)
    return _conv1_lin_tc(h1, aggp1, degp3, W1l, W1r, b1.reshape(1, D),
                         Wlin, blin.reshape(1, D))


# speedup vs baseline: 11.7072x; 1.0407x over previous
import functools

import jax
import jax.numpy as jnp
from jax import lax
from jax.experimental import pallas as pl
from jax.experimental.pallas import tpu as pltpu
from jax.experimental.pallas import tpu_sc as plsc

N = 10000
E = 320000
D = 128

NC = 2
NS = 16
L = 16
NW = NC * NS
EW = E // NW
CH = 125
NCHUNK = EW // CH
NHALF = NCHUNK // 2
ONESB = 128
ROFF = 624
RLEN = 640
ZR = 40

_mesh = plsc.VectorSubcoreMesh(core_axis_name="c", subcore_axis_name="s")


def _make_agg(want_deg):
    out_types = [jax.ShapeDtypeStruct((NC, N, D), jnp.float32)]
    if want_deg:
        out_types.append(jax.ShapeDtypeStruct((NC * N,), jnp.float32))
    scratch = [
        pltpu.VMEM((NHALF, CH), jnp.int32),
        pltpu.VMEM((NHALF, CH), jnp.int32),
        pltpu.VMEM((CH, D), jnp.float32),
        pltpu.VMEM((CH, D), jnp.float32),
        pltpu.VMEM((ZR, D), jnp.float32),
        pltpu.VMEM_SHARED((N, D), jnp.float32),
        pltpu.SemaphoreType.DMA,
        pltpu.SemaphoreType.DMA,
        pltpu.SemaphoreType.DMA,
        pltpu.SemaphoreType.DMA,
        pltpu.SemaphoreType.DMA,
    ]
    if want_deg:
        scratch += [
            pltpu.VMEM((RLEN,), jnp.float32),
            pltpu.VMEM((ONESB,), jnp.float32),
            pltpu.VMEM_SHARED((N,), jnp.float32),
            pltpu.SemaphoreType.DMA,
        ]

    @functools.partial(
        pl.kernel, mesh=_mesh,
        out_type=tuple(out_types) if want_deg else out_types[0],
        scratch_types=scratch)
    def agg_kernel(edge_hbm, x_hbm, *refs):
        if want_deg:
            (agg_out, deg_out, src_v, dst_v, rows_a, rows_b, z2d, acc_sh,
             gsem_a, gsem_b, ssem_a, ssem_b, zsem, zdeg, ones_v, deg_sh,
             dsem) = refs
        else:
            (agg_out, src_v, dst_v, rows_a, rows_b, z2d, acc_sh,
             gsem_a, gsem_b, ssem_a, ssem_b, zsem) = refs
        cid = lax.axis_index("c")
        sid = lax.axis_index("s")
        wid = sid * NC + cid
        rows = (rows_a, rows_b)
        gsem = (gsem_a, gsem_b)
        ssem = (ssem_a, ssem_b)

        def gather(j, p):
            return pltpu.make_async_copy(x_hbm.at[src_v.at[j]], rows[p],
                                         gsem[p])

        def scatter(j, p):
            return pltpu.make_async_copy(rows[p], acc_sh.at[dst_v.at[j]],
                                         ssem[p])

        def zfill(i, _):
            z2d[i // (D // L), pl.ds((i % (D // L)) * L, L)] = jnp.zeros(
                (L,), jnp.float32)
            return 0
        lax.fori_loop(0, ZR * (D // L), zfill, 0)
        if want_deg:
            def zdfill(i, _):
                zdeg[pl.ds(i * L, L)] = jnp.zeros((L,), jnp.float32)
                return 0
            lax.fori_loop(0, RLEN // L, zdfill, 0)
            for q in range(ONESB // L):
                ones_v[pl.ds(q * L, L)] = jnp.ones((L,), jnp.float32)

        roff = pl.multiple_of(sid * ROFF, 8)
        def zcopy(q):
            return pltpu.make_async_copy(z2d, acc_sh.at[pl.ds(
                pl.multiple_of(sid * ROFF + q * ZR, 8), ZR)], zsem)
        def zacc_start(q, _):
            zcopy(q).start()
            return 0
        lax.fori_loop(0, RLEN // ZR, zacc_start, 0)
        if want_deg:
            pltpu.make_async_copy(zdeg, deg_sh.at[pl.ds(roff, RLEN)],
                                  zsem).start()

        pltpu.sync_copy(edge_hbm.at[0, wid, 0], src_v)
        pltpu.sync_copy(edge_hbm.at[1, wid, 0], dst_v)
        gather(0, 0).start()
        gather(1, 1).start()

        def zacc_wait(q, _):
            zcopy(q).wait()
            return 0
        lax.fori_loop(0, RLEN // ZR, zacc_wait, 0)
        if want_deg:
            pltpu.make_async_copy(zdeg, deg_sh.at[pl.ds(roff, RLEN)],
                                  zsem).wait()

        plsc.subcore_barrier()

        def scat_deg(j):
            if want_deg:
                pltpu.make_async_copy(
                    ones_v.at[pl.ds(0, CH)], deg_sh.at[dst_v.at[j]],
                    dsem).start(add=True)

        def chunk2(t, _):
            j = t * 2
            gather(j, 0).wait()
            scatter(j, 0).start(add=True)
            scat_deg(j)
            gather(j + 1, 1).wait()
            scatter(j + 1, 1).start(add=True)
            scat_deg(j + 1)
            scatter(j, 0).wait()
            gather(j + 2, 0).start()
            scatter(j + 1, 1).wait()
            gather(j + 3, 1).start()
            return 0

        for h in range(2):
            if h > 0:
                pltpu.sync_copy(edge_hbm.at[0, wid, h], src_v)
                pltpu.sync_copy(edge_hbm.at[1, wid, h], dst_v)
                gather(0, 0).start()
                gather(1, 1).start()
            lax.fori_loop(0, NHALF // 2 - 1, chunk2, 0)
            j = NHALF - 2
            gather(j, 0).wait()
            scatter(j, 0).start(add=True)
            scat_deg(j)
            gather(j + 1, 1).wait()
            scatter(j + 1, 1).start(add=True)
            scat_deg(j + 1)
            scatter(j, 0).wait()
            scatter(j + 1, 1).wait()
            if want_deg:
                def ddrain(i, _):
                    pltpu.make_async_copy(
                        ones_v.at[pl.ds(0, CH)], deg_sh.at[dst_v.at[i]],
                        dsem).wait()
                    return 0
                lax.fori_loop(0, NHALF, ddrain, 0)

        plsc.subcore_barrier()

        pltpu.sync_copy(acc_sh.at[pl.ds(roff, RLEN)],
                        agg_out.at[cid, pl.ds(roff, RLEN)])
        if want_deg:
            pltpu.sync_copy(deg_sh.at[pl.ds(roff, RLEN)], zdeg)
            pltpu.sync_copy(zdeg, deg_out.at[pl.ds(pl.multiple_of(
                cid * N + sid * ROFF, 8), RLEN)])

    return agg_kernel


_agg_deg = _make_agg(True)
_agg = _make_agg(False)

R = 2000


def _mm_t(a, w):
    return lax.dot_general(a, w, (((1,), (1,)), ((), ())),
                           preferred_element_type=jnp.float32)


def _conv0_body(x_ref, a_ref, d_ref, wl_ref, wr_ref, b_ref, o_ref):
    inv = 1.0 / jnp.maximum(d_ref[0] + d_ref[1], 1.0)
    mean = (a_ref[0] + a_ref[1]) * inv
    h = _mm_t(x_ref[...], wl_ref[...]) + _mm_t(mean, wr_ref[...]) + b_ref[...]
    o_ref[...] = jnp.maximum(h, 0.0)


def _conv1_lin_body(x_ref, a_ref, d_ref, wl_ref, wr_ref, b_ref, wlin_ref,
                    blin_ref, o_ref):
    inv = 1.0 / jnp.maximum(d_ref[0] + d_ref[1], 1.0)
    mean = (a_ref[0] + a_ref[1]) * inv
    h = _mm_t(x_ref[...], wl_ref[...]) + _mm_t(mean, wr_ref[...]) + b_ref[...]
    o_ref[...] = _mm_t(h, wlin_ref[...]) + blin_ref[...]


_row_spec = pl.BlockSpec((R, D), lambda i: (i, 0))
_agg_spec = pl.BlockSpec((2, R, D), lambda i: (0, i, 0))
_deg_spec = pl.BlockSpec((2, R, 1), lambda i: (0, i, 0))
_w_spec = pl.BlockSpec((D, D), lambda i: (0, 0))
_b_spec = pl.BlockSpec((1, D), lambda i: (0, 0))


def _conv0_tc(x, aggp, degp3, Wl, Wr, b):
    return pl.pallas_call(
        _conv0_body,
        grid=(N // R,),
        in_specs=[_row_spec, _agg_spec, _deg_spec, _w_spec, _w_spec, _b_spec],
        out_specs=_row_spec,
        out_shape=jax.ShapeDtypeStruct((N, D), jnp.float32),
    )(x, aggp, degp3, Wl, Wr, b)


def _conv1_lin_tc(h1, aggp, degp3, Wl, Wr, b, Wlin, blin):
    return pl.pallas_call(
        _conv1_lin_body,
        grid=(N // R,),
        in_specs=[_row_spec, _agg_spec, _deg_spec, _w_spec, _w_spec, _b_spec,
                  _w_spec, _b_spec],
        out_specs=_row_spec,
        out_shape=jax.ShapeDtypeStruct((N, D), jnp.float32),
    )(h1, aggp, degp3, Wl, Wr, b, Wlin, blin)


def kernel(x, edge_index, W0l, W0r, b0, W1l, W1r, b1, Wlin, blin):
    edge_r = edge_index.reshape(2, NW, 2, NHALF, CH)
    aggp0, degp = _agg_deg(edge_r, x)
    degp3 = degp.reshape(NC, N, 1)
    h1 = _conv0_tc(x, aggp0, degp3, W0l, W0r, b0.reshape(1, D))
    aggp1 = _agg(edge_r, h1)
    return _conv1_lin_tc(h1, aggp1, degp3, W1l, W1r, b1.reshape(1, D),
                         Wlin, blin.reshape(1, D))
